# baseline (speedup 1.0000x reference)
import functools

import jax
import jax.numpy as jnp
from jax import lax
from jax.experimental import pallas as pl
from jax.experimental.pallas import tpu as pltpu
from jax.experimental.pallas import tpu_sc as plsc

N, TOPK, D, DFF, E = 2048, 2, 1024, 4096, 8
NK = N * TOPK
BR = 512
_BRSH = 9
CAP = NK + E * BR
T = CAP // BR
BF = 512
NF = DFF // BF

_mesh = plsc.VectorSubcoreMesh(core_axis_name="c", subcore_axis_name="s")
_sc_params = pltpu.CompilerParams(needs_layout_passes=False)

_NT = 16
_CH = NK // _NT
_NV = _CH // 16
_ZCH = CAP // _NT


@functools.partial(
    pl.kernel,
    out_type=(
        jax.ShapeDtypeStruct((NK,), jnp.int32),
        jax.ShapeDtypeStruct((CAP,), jnp.int32),
        jax.ShapeDtypeStruct((16,), jnp.int32),
        jax.ShapeDtypeStruct((32,), jnp.int32),
    ),
    mesh=_mesh,
    scratch_types=(
        pltpu.VMEM((_CH,), jnp.int32),
        pltpu.VMEM((_CH,), jnp.int32),
        pltpu.VMEM((16,), jnp.int32),
        pltpu.VMEM((16,), jnp.int32),
        pltpu.VMEM((_CH,), jnp.int32),
        pltpu.VMEM((CAP,), jnp.int32),
        pltpu.VMEM((_ZCH,), jnp.int32),
        pltpu.VMEM((_ZCH,), jnp.int32),
        pltpu.VMEM((_NT, 16), jnp.int32),
        pltpu.VMEM((16,), jnp.int32),
        pltpu.VMEM((32,), jnp.int32),
        pltpu.VMEM_SHARED((_NT, 16), jnp.int32),
        pltpu.VMEM_SHARED((_NT, CAP), jnp.int32),
    ),
    compiler_params=_sc_params,
)
def _sc_route(eflat, dst_hbm, tsrc_hbm, counts_hbm, te_hbm,
              ev, lrank, histv, basev, dstbuf, tloc, tmpbuf, accbuf,
              allhist, cntbuf, tebuf, shist, tshared):
    c = lax.axis_index("c")
    w = lax.axis_index("s")
    lane = lax.iota(jnp.int32, 16)
    on0 = c == 0

    @pl.when(on0)
    def _phase1():
        pltpu.sync_copy(eflat.at[pl.ds(w * _CH, _CH)], ev)

        histv[...] = jnp.zeros((16,), jnp.int32)
        h = jnp.zeros((16,), jnp.int32)
        for i in range(_NV):
            v = ev[pl.ds(i * 16, 16)]
            base = plsc.load_gather(histv, [v])
            rankv = jnp.zeros((16,), jnp.int32)
            for ex in range(E):
                m = v == ex
                mi = m.astype(jnp.int32)
                cs = plsc.cumsum(mi)
                rankv = rankv + jnp.where(m, cs - 1, 0)
                h = h + jnp.where(lane == ex, jnp.sum(mi), 0)
            lrank[pl.ds(i * 16, 16)] = base + rankv
            histv[...] = h
        pltpu.sync_copy(histv, shist.at[w])

    plsc.subcore_barrier()

    @pl.when(on0)
    def _phase2():
        pltpu.sync_copy(shist, allhist)

        countsv = jnp.zeros((16,), jnp.int32)
        startv = jnp.zeros((16,), jnp.int32)
        for wp in range(_NT):
            row = allhist[wp, :]
            countsv = countsv + row
            startv = startv + jnp.where(wp < w, row, 0)

        rc = ((countsv + (BR - 1)) >> _BRSH) << _BRSH
        po_incl = plsc.cumsum(rc)
        po = po_incl - rc
        basev[...] = po + startv

        for i in range(CAP // 16):
            tloc[pl.ds(i * 16, 16)] = jnp.zeros((16,), jnp.int32)
        for i in range(_NV):
            v = ev[pl.ds(i * 16, 16)]
            b = plsc.load_gather(basev, [v])
            dv = b + lrank[pl.ds(i * 16, 16)]
            dstbuf[pl.ds(i * 16, 16)] = dv
            tv = (w * _CH + i * 16 + lane) >> 1
            plsc.store_scatter(tloc, [dv], tv)
        pltpu.sync_copy(dstbuf, dst_hbm.at[pl.ds(w * _CH, _CH)])
        pltpu.sync_copy(tloc, tshared.at[w])

        @pl.when(w == 0)
        def _tile0():
            cntbuf[...] = countsv
            pltpu.sync_copy(cntbuf, counts_hbm)
            for t0 in (0, 16):
                tb = jnp.zeros((16,), jnp.int32)
                tvec = (lane + t0) * BR
                for ex in range(E):
                    pe = jnp.sum(jnp.where(lane == ex, po_incl, 0))
                    tb = tb + (tvec >= pe).astype(jnp.int32)
                tebuf[pl.ds(t0, 16)] = jnp.minimum(tb, E - 1)
            pltpu.sync_copy(tebuf, te_hbm)

    plsc.subcore_barrier()

    @pl.when(on0)
    def _phase3():
        acc = [jnp.zeros((16,), jnp.int32) for _ in range(_ZCH // 16)]
        for wp in range(_NT):
            pltpu.sync_copy(tshared.at[wp, pl.ds(w * _ZCH, _ZCH)], tmpbuf)
            for i in range(_ZCH // 16):
                acc[i] = acc[i] + tmpbuf[pl.ds(i * 16, 16)]
        for i in range(_ZCH // 16):
            accbuf[pl.ds(i * 16, 16)] = acc[i]
        pltpu.sync_copy(accbuf, tsrc_hbm.at[pl.ds(w * _ZCH, _ZCH)])


_GROWS = CAP // 32
_GC = 64


@functools.partial(
    pl.kernel,
    out_type=jax.ShapeDtypeStruct((CAP, D), jnp.float32),
    mesh=_mesh,
    scratch_types=(
        pltpu.VMEM((_GC,), jnp.int32),
        pltpu.VMEM((_GC, D), jnp.float32),
        pltpu.SemaphoreType.DMA,
    ),
    compiler_params=_sc_params,
)
def _sc_gather(x_hbm, tsrc_hbm, xs_hbm, idxv, rows, sem):
    wid = lax.axis_index("s") * 2 + lax.axis_index("c")
    for ch in range(_GROWS // _GC):
        base = wid * _GROWS + ch * _GC
        pltpu.sync_copy(tsrc_hbm.at[pl.ds(base, _GC)], idxv)
        pltpu.async_copy(x_hbm.at[idxv], rows, sem).wait()
        pltpu.sync_copy(rows, xs_hbm.at[pl.ds(base, _GC)])


_CT = N // 32
_CC = 32


@functools.partial(
    pl.kernel,
    out_type=(
        jax.ShapeDtypeStruct((N, D), jnp.float32),
        jax.ShapeDtypeStruct((N, D), jnp.float32),
    ),
    mesh=_mesh,
    scratch_types=(
        pltpu.VMEM((2 * _CT,), jnp.int32),
        pltpu.VMEM((_CC,), jnp.int32),
        pltpu.VMEM((_CC, D), jnp.float32),
        pltpu.SemaphoreType.DMA,
    ),
    compiler_params=_sc_params,
)
def _sc_pick(o_hbm, dst_hbm, g0_hbm, g1_hbm, dstloc, idxv, rows, sem):
    wid = lax.axis_index("s") * 2 + lax.axis_index("c")
    lane = lax.iota(jnp.int32, 16)
    pltpu.sync_copy(dst_hbm.at[pl.ds(wid * 2 * _CT, 2 * _CT)], dstloc)
    for ch in range(_CT // _CC):
        for slot in range(2):
            for j in range(_CC // 16):
                lt0 = ch * _CC + j * 16
                idxv[pl.ds(j * 16, 16)] = plsc.load_gather(
                    dstloc, [lane * 2 + (lt0 * 2 + slot)])
            pltpu.async_copy(o_hbm.at[idxv], rows, sem).wait()
            gh = g0_hbm if slot == 0 else g1_hbm
            pltpu.sync_copy(rows, gh.at[pl.ds(wid * _CT + ch * _CC, _CC)])


def _mlp_body(te_ref, xs_ref, w1_ref, w3_ref, w2_ref, o_ref):
    f = pl.program_id(1)

    @pl.when(f == 0)
    def _():
        o_ref[...] = jnp.zeros_like(o_ref)

    xb = xs_ref[...].astype(jnp.bfloat16)
    w1b = w1_ref[0].astype(jnp.bfloat16)
    w3b = w3_ref[0].astype(jnp.bfloat16)
    a1 = jnp.dot(xb, w1b, preferred_element_type=jnp.float32)
    a3 = jnp.dot(xb, w3b, preferred_element_type=jnp.float32)
    h = (jax.nn.silu(a1) * a3).astype(jnp.bfloat16)
    w2b = w2_ref[0].astype(jnp.bfloat16)
    o_ref[...] += jnp.dot(h, w2b, preferred_element_type=jnp.float32)


def _grouped_mlp(te, xs, w1, w3, w2):
    grid_spec = pltpu.PrefetchScalarGridSpec(
        num_scalar_prefetch=1,
        grid=(T, NF),
        in_specs=[
            pl.BlockSpec((BR, D), lambda t, f, te_r: (t, 0)),
            pl.BlockSpec((1, D, BF), lambda t, f, te_r: (te_r[t], 0, f)),
            pl.BlockSpec((1, D, BF), lambda t, f, te_r: (te_r[t], 0, f)),
            pl.BlockSpec((1, BF, D), lambda t, f, te_r: (te_r[t], f, 0)),
        ],
        out_specs=pl.BlockSpec((BR, D), lambda t, f, te_r: (t, 0)),
    )
    return pl.pallas_call(
        _mlp_body,
        grid_spec=grid_spec,
        out_shape=jax.ShapeDtypeStruct((CAP, D), jnp.float32),
        compiler_params=pltpu.CompilerParams(
            dimension_semantics=("arbitrary", "arbitrary")),
    )(te, xs, w1, w3, w2)


def _comb_body(g0_ref, g1_ref, e0_ref, e1_ref, y_ref):
    y_ref[...] = e0_ref[...] * g0_ref[...] + e1_ref[...] * g1_ref[...]


def _combine(g0, g1, ew):
    return pl.pallas_call(
        _comb_body,
        grid=(N // BR,),
        in_specs=[
            pl.BlockSpec((BR, D), lambda i: (i, 0)),
            pl.BlockSpec((BR, D), lambda i: (i, 0)),
            pl.BlockSpec((BR, 1), lambda i: (i, 0)),
            pl.BlockSpec((BR, 1), lambda i: (i, 0)),
        ],
        out_specs=pl.BlockSpec((BR, D), lambda i: (i, 0)),
        out_shape=jax.ShapeDtypeStruct((N, D), jnp.float32),
    )(g0, g1, ew[:, :1], ew[:, 1:])


def kernel(x, expert_weights, expert_indices, w1, w2, w3):
    eflat = expert_indices.reshape(-1).astype(jnp.int32)
    dst, tsrc, counts16, te = _sc_route(eflat)
    xs = _sc_gather(x, tsrc)
    o = _grouped_mlp(te, xs, w1, w3, w2)
    g0, g1 = _sc_pick(o, dst)
    y = _combine(g0, g1, expert_weights)
    return y, counts16[:8]


# speedup vs baseline: 1.0045x; 1.0045x over previous
import functools

import jax
import jax.numpy as jnp
from jax import lax
from jax.experimental import pallas as pl
from jax.experimental.pallas import tpu as pltpu
from jax.experimental.pallas import tpu_sc as plsc

N, TOPK, D, DFF, E = 2048, 2, 1024, 4096, 8
NK = N * TOPK
BR = 512
_BRSH = 9
CAP = NK + E * BR
T = CAP // BR
BF = 512
NF = DFF // BF

_mesh = plsc.VectorSubcoreMesh(core_axis_name="c", subcore_axis_name="s")
_sc_params = pltpu.CompilerParams(needs_layout_passes=False)

_NT = 16
_CH = NK // _NT
_NV = _CH // 16
_ZCH = CAP // _NT
_GROWS = CAP // 32
_GC = 32
_NGC = _GROWS // _GC


@functools.partial(
    pl.kernel,
    out_type=(
        jax.ShapeDtypeStruct((NK,), jnp.int32),
        jax.ShapeDtypeStruct((16,), jnp.int32),
        jax.ShapeDtypeStruct((32,), jnp.int32),
        jax.ShapeDtypeStruct((CAP, D), jnp.float32),
    ),
    mesh=_mesh,
    scratch_types=(
        pltpu.VMEM((_CH,), jnp.int32),
        pltpu.VMEM((_CH,), jnp.int32),
        pltpu.VMEM((16,), jnp.int32),
        pltpu.VMEM((16,), jnp.int32),
        pltpu.VMEM((_CH,), jnp.int32),
        pltpu.VMEM((CAP,), jnp.int32),
        pltpu.VMEM((_ZCH,), jnp.int32),
        pltpu.VMEM((_ZCH,), jnp.int32),
        pltpu.VMEM((_NT, 16), jnp.int32),
        pltpu.VMEM((16,), jnp.int32),
        pltpu.VMEM((32,), jnp.int32),
        pltpu.VMEM((_GROWS,), jnp.int32),
        pltpu.VMEM((_GC, D), jnp.float32),
        pltpu.VMEM((_GC, D), jnp.float32),
        pltpu.VMEM_SHARED((_NT, 16), jnp.int32),
        pltpu.VMEM_SHARED((_NT, CAP), jnp.int32),
        pltpu.VMEM_SHARED((CAP,), jnp.int32),
        pltpu.SemaphoreType.DMA,
        pltpu.SemaphoreType.DMA,
        pltpu.SemaphoreType.DMA,
        pltpu.SemaphoreType.DMA,
    ),
    compiler_params=_sc_params,
)
def _sc_route(eflat, x_hbm, dst_hbm, counts_hbm, te_hbm, xs_hbm,
              ev, lrank, histv, basev, dstbuf, tloc, tmpbuf, accbuf,
              allhist, cntbuf, tebuf, gidx, grows0, grows1,
              shist, tshared, tsrc_sh, gsem0, gsem1, wsem0, wsem1):
    c = lax.axis_index("c")
    w = lax.axis_index("s")
    lane = lax.iota(jnp.int32, 16)

    pltpu.sync_copy(eflat.at[pl.ds(w * _CH, _CH)], ev)
    histv[...] = jnp.zeros((16,), jnp.int32)
    h = jnp.zeros((16,), jnp.int32)
    for i in range(_NV):
        v = ev[pl.ds(i * 16, 16)]
        base = plsc.load_gather(histv, [v])
        rankv = jnp.zeros((16,), jnp.int32)
        for ex in range(E):
            m = v == ex
            mi = m.astype(jnp.int32)
            cs = plsc.cumsum(mi)
            rankv = rankv + jnp.where(m, cs - 1, 0)
            h = h + jnp.where(lane == ex, jnp.sum(mi), 0)
        lrank[pl.ds(i * 16, 16)] = base + rankv
        histv[...] = h
    pltpu.sync_copy(histv, shist.at[w])

    plsc.subcore_barrier()

    pltpu.sync_copy(shist, allhist)
    countsv = jnp.zeros((16,), jnp.int32)
    startv = jnp.zeros((16,), jnp.int32)
    for wp in range(_NT):
        row = allhist[wp, :]
        countsv = countsv + row
        startv = startv + jnp.where(wp < w, row, 0)

    rc = ((countsv + (BR - 1)) >> _BRSH) << _BRSH
    po_incl = plsc.cumsum(rc)
    po = po_incl - rc
    basev[...] = po + startv

    for i in range(CAP // 16):
        tloc[pl.ds(i * 16, 16)] = jnp.zeros((16,), jnp.int32)
    for i in range(_NV):
        v = ev[pl.ds(i * 16, 16)]
        b = plsc.load_gather(basev, [v])
        dv = b + lrank[pl.ds(i * 16, 16)]
        dstbuf[pl.ds(i * 16, 16)] = dv
        tv = (w * _CH + i * 16 + lane) >> 1
        plsc.store_scatter(tloc, [dv], tv)
    pltpu.sync_copy(tloc, tshared.at[w])

    @pl.when(c == 0)
    def _hbm_meta():
        pltpu.sync_copy(dstbuf, dst_hbm.at[pl.ds(w * _CH, _CH)])

        @pl.when(w == 0)
        def _tile0():
            cntbuf[...] = countsv
            pltpu.sync_copy(cntbuf, counts_hbm)
            for t0 in (0, 16):
                tb = jnp.zeros((16,), jnp.int32)
                tvec = (lane + t0) * BR
                for ex in range(E):
                    pe = jnp.sum(jnp.where(lane == ex, po_incl, 0))
                    tb = tb + (tvec >= pe).astype(jnp.int32)
                tebuf[pl.ds(t0, 16)] = jnp.minimum(tb, E - 1)
            pltpu.sync_copy(tebuf, te_hbm)

    plsc.subcore_barrier()

    acc = [jnp.zeros((16,), jnp.int32) for _ in range(_ZCH // 16)]
    for wp in range(_NT):
        pltpu.sync_copy(tshared.at[wp, pl.ds(w * _ZCH, _ZCH)], tmpbuf)
        for i in range(_ZCH // 16):
            acc[i] = acc[i] + tmpbuf[pl.ds(i * 16, 16)]
    for i in range(_ZCH // 16):
        accbuf[pl.ds(i * 16, 16)] = acc[i]
    pltpu.sync_copy(accbuf, tsrc_sh.at[pl.ds(w * _ZCH, _ZCH)])

    plsc.subcore_barrier()

    wid = w * 2 + c
    base0 = wid * _GROWS
    pltpu.sync_copy(tsrc_sh.at[pl.ds(base0, _GROWS)], gidx)
    bufs = (grows0, grows1)
    gsems = (gsem0, gsem1)
    wsems = (wsem0, wsem1)
    gd = [None, None]
    wd = [None, None]
    for ch in range(_NGC + 1):
        b = ch & 1
        if ch < _NGC:
            if wd[b] is not None:
                wd[b].wait()
            gd[b] = pltpu.async_copy(
                x_hbm.at[gidx.at[pl.ds(ch * _GC, _GC)]], bufs[b], gsems[b])
        if ch >= 1:
            pb = (ch - 1) & 1
            gd[pb].wait()
            wd[pb] = pltpu.async_copy(
                bufs[pb], xs_hbm.at[pl.ds(base0 + (ch - 1) * _GC, _GC)],
                wsems[pb])
    wd[(_NGC - 1) & 1].wait()
    wd[_NGC & 1].wait()


_CT = N // 32
_CC = 32


@functools.partial(
    pl.kernel,
    out_type=(
        jax.ShapeDtypeStruct((N, D), jnp.float32),
        jax.ShapeDtypeStruct((N, D), jnp.float32),
    ),
    mesh=_mesh,
    scratch_types=(
        pltpu.VMEM((2 * _CT,), jnp.int32),
        pltpu.VMEM((_CC,), jnp.int32),
        pltpu.VMEM((_CC, D), jnp.float32),
        pltpu.SemaphoreType.DMA,
    ),
    compiler_params=_sc_params,
)
def _sc_pick(o_hbm, dst_hbm, g0_hbm, g1_hbm, dstloc, idxv, rows, sem):
    wid = lax.axis_index("s") * 2 + lax.axis_index("c")
    lane = lax.iota(jnp.int32, 16)
    pltpu.sync_copy(dst_hbm.at[pl.ds(wid * 2 * _CT, 2 * _CT)], dstloc)
    for ch in range(_CT // _CC):
        for slot in range(2):
            for j in range(_CC // 16):
                lt0 = ch * _CC + j * 16
                idxv[pl.ds(j * 16, 16)] = plsc.load_gather(
                    dstloc, [lane * 2 + (lt0 * 2 + slot)])
            pltpu.async_copy(o_hbm.at[idxv], rows, sem).wait()
            gh = g0_hbm if slot == 0 else g1_hbm
            pltpu.sync_copy(rows, gh.at[pl.ds(wid * _CT + ch * _CC, _CC)])


def _mlp_body(te_ref, xs_ref, w1_ref, w3_ref, w2_ref, o_ref):
    f = pl.program_id(1)

    @pl.when(f == 0)
    def _():
        o_ref[...] = jnp.zeros_like(o_ref)

    xb = xs_ref[...].astype(jnp.bfloat16)
    w1b = w1_ref[0].astype(jnp.bfloat16)
    w3b = w3_ref[0].astype(jnp.bfloat16)
    a1 = jnp.dot(xb, w1b, preferred_element_type=jnp.float32)
    a3 = jnp.dot(xb, w3b, preferred_element_type=jnp.float32)
    h = (jax.nn.silu(a1) * a3).astype(jnp.bfloat16)
    w2b = w2_ref[0].astype(jnp.bfloat16)
    o_ref[...] += jnp.dot(h, w2b, preferred_element_type=jnp.float32)


def _grouped_mlp(te, xs, w1, w3, w2):
    grid_spec = pltpu.PrefetchScalarGridSpec(
        num_scalar_prefetch=1,
        grid=(T, NF),
        in_specs=[
            pl.BlockSpec((BR, D), lambda t, f, te_r: (t, 0)),
            pl.BlockSpec((1, D, BF), lambda t, f, te_r: (te_r[t], 0, f)),
            pl.BlockSpec((1, D, BF), lambda t, f, te_r: (te_r[t], 0, f)),
            pl.BlockSpec((1, BF, D), lambda t, f, te_r: (te_r[t], f, 0)),
        ],
        out_specs=pl.BlockSpec((BR, D), lambda t, f, te_r: (t, 0)),
    )
    return pl.pallas_call(
        _mlp_body,
        grid_spec=grid_spec,
        out_shape=jax.ShapeDtypeStruct((CAP, D), jnp.float32),
        compiler_params=pltpu.CompilerParams(
            dimension_semantics=("arbitrary", "arbitrary")),
    )(te, xs, w1, w3, w2)


def _comb_body(g0_ref, g1_ref, e0_ref, e1_ref, y_ref):
    y_ref[...] = e0_ref[...] * g0_ref[...] + e1_ref[...] * g1_ref[...]


def _combine(g0, g1, ew):
    return pl.pallas_call(
        _comb_body,
        grid=(N // BR,),
        in_specs=[
            pl.BlockSpec((BR, D), lambda i: (i, 0)),
            pl.BlockSpec((BR, D), lambda i: (i, 0)),
            pl.BlockSpec((BR, 1), lambda i: (i, 0)),
            pl.BlockSpec((BR, 1), lambda i: (i, 0)),
        ],
        out_specs=pl.BlockSpec((BR, D), lambda i: (i, 0)),
        out_shape=jax.ShapeDtypeStruct((N, D), jnp.float32),
    )(g0, g1, ew[:, :1], ew[:, 1:])


def kernel(x, expert_weights, expert_indices, w1, w2, w3):
    eflat = expert_indices.reshape(-1).astype(jnp.int32)
    dst, counts16, te, xs = _sc_route(eflat, x)
    o = _grouped_mlp(te, xs, w1, w3, w2)
    g0, g1 = _sc_pick(o, dst)
    y = _combine(g0, g1, expert_weights)
    return y, counts16[:8]


# speedup vs baseline: 1.4424x; 1.4359x over previous
import functools

import jax
import jax.numpy as jnp
from jax import lax
from jax.experimental import pallas as pl
from jax.experimental.pallas import tpu as pltpu
from jax.experimental.pallas import tpu_sc as plsc

N, TOPK, D, DFF, E = 2048, 2, 1024, 4096, 8
NK = N * TOPK
BR = 512
_BRSH = 9
CAP = NK + E * BR
T = CAP // BR
BF = 512
NF = DFF // BF

_mesh = plsc.VectorSubcoreMesh(core_axis_name="c", subcore_axis_name="s")
_sc_params = pltpu.CompilerParams(needs_layout_passes=False)

_NT = 16
_CH = NK // _NT
_NV = _CH // 16
_ZCH = CAP // _NT
@functools.partial(
    pl.kernel,
    out_type=(
        jax.ShapeDtypeStruct((NK,), jnp.int32),
        jax.ShapeDtypeStruct((CAP,), jnp.int32),
        jax.ShapeDtypeStruct((16,), jnp.int32),
        jax.ShapeDtypeStruct((32,), jnp.int32),
    ),
    mesh=_mesh,
    scratch_types=(
        pltpu.VMEM((_CH,), jnp.int32),
        pltpu.VMEM((_CH,), jnp.int32),
        pltpu.VMEM((16,), jnp.int32),
        pltpu.VMEM((16,), jnp.int32),
        pltpu.VMEM((_CH,), jnp.int32),
        pltpu.VMEM((CAP,), jnp.int32),
        pltpu.VMEM((_ZCH,), jnp.int32),
        pltpu.VMEM((_ZCH,), jnp.int32),
        pltpu.VMEM((_NT, 16), jnp.int32),
        pltpu.VMEM((16,), jnp.int32),
        pltpu.VMEM((32,), jnp.int32),
        pltpu.VMEM_SHARED((_NT, 16), jnp.int32),
        pltpu.VMEM_SHARED((_NT, CAP), jnp.int32),
    ),
    compiler_params=_sc_params,
)
def _sc_route(eflat, dst_hbm, tsrc_hbm, counts_hbm, te_hbm,
              ev, lrank, histv, basev, dstbuf, tloc, tmpbuf, accbuf,
              allhist, cntbuf, tebuf, shist, tshared):
    c = lax.axis_index("c")
    w = lax.axis_index("s")
    lane = lax.iota(jnp.int32, 16)

    pltpu.sync_copy(eflat.at[pl.ds(w * _CH, _CH)], ev)
    histv[...] = jnp.zeros((16,), jnp.int32)
    h = jnp.zeros((16,), jnp.int32)
    for i in range(_NV):
        v = ev[pl.ds(i * 16, 16)]
        base = plsc.load_gather(histv, [v])
        rankv = jnp.zeros((16,), jnp.int32)
        for ex in range(E):
            m = v == ex
            mi = m.astype(jnp.int32)
            cs = plsc.cumsum(mi)
            rankv = rankv + jnp.where(m, cs - 1, 0)
            h = h + jnp.where(lane == ex, jnp.sum(mi), 0)
        lrank[pl.ds(i * 16, 16)] = base + rankv
        histv[...] = h
    pltpu.sync_copy(histv, shist.at[w])

    plsc.subcore_barrier()

    pltpu.sync_copy(shist, allhist)
    countsv = jnp.zeros((16,), jnp.int32)
    startv = jnp.zeros((16,), jnp.int32)
    for wp in range(_NT):
        row = allhist[wp, :]
        countsv = countsv + row
        startv = startv + jnp.where(wp < w, row, 0)

    rc = ((countsv + (BR - 1)) >> _BRSH) << _BRSH
    po_incl = plsc.cumsum(rc)
    po = po_incl - rc
    basev[...] = po + startv

    for i in range(CAP // 16):
        tloc[pl.ds(i * 16, 16)] = jnp.zeros((16,), jnp.int32)
    for i in range(_NV):
        v = ev[pl.ds(i * 16, 16)]
        b = plsc.load_gather(basev, [v])
        dv = b + lrank[pl.ds(i * 16, 16)]
        dstbuf[pl.ds(i * 16, 16)] = dv
        tv = (w * _CH + i * 16 + lane) >> 1
        plsc.store_scatter(tloc, [dv], tv)
    pltpu.sync_copy(tloc, tshared.at[w])

    @pl.when(c == 0)
    def _hbm_meta():
        pltpu.sync_copy(dstbuf, dst_hbm.at[pl.ds(w * _CH, _CH)])

        @pl.when(w == 0)
        def _tile0():
            cntbuf[...] = countsv
            pltpu.sync_copy(cntbuf, counts_hbm)
            for t0 in (0, 16):
                tb = jnp.zeros((16,), jnp.int32)
                tvec = (lane + t0) * BR
                for ex in range(E):
                    pe = jnp.sum(jnp.where(lane == ex, po_incl, 0))
                    tb = tb + (tvec >= pe).astype(jnp.int32)
                tebuf[pl.ds(t0, 16)] = jnp.minimum(tb, E - 1)
            pltpu.sync_copy(tebuf, te_hbm)

    plsc.subcore_barrier()

    acc = [jnp.zeros((16,), jnp.int32) for _ in range(_ZCH // 16)]
    for wp in range(_NT):
        pltpu.sync_copy(tshared.at[wp, pl.ds(w * _ZCH, _ZCH)], tmpbuf)
        for i in range(_ZCH // 16):
            acc[i] = acc[i] + tmpbuf[pl.ds(i * 16, 16)]
    for i in range(_ZCH // 16):
        accbuf[pl.ds(i * 16, 16)] = acc[i]

    @pl.when(c == 0)
    def _tsrc_out():
        pltpu.sync_copy(accbuf, tsrc_hbm.at[pl.ds(w * _ZCH, _ZCH)])


_CT = N // 32
_CC = 32


@functools.partial(
    pl.kernel,
    out_type=(
        jax.ShapeDtypeStruct((N, D), jnp.float32),
        jax.ShapeDtypeStruct((N, D), jnp.float32),
    ),
    mesh=_mesh,
    scratch_types=(
        pltpu.VMEM((2 * _CT,), jnp.int32),
        pltpu.VMEM((_CC,), jnp.int32),
        pltpu.VMEM((_CC, D), jnp.float32),
        pltpu.SemaphoreType.DMA,
    ),
    compiler_params=_sc_params,
)
def _sc_pick(o_hbm, dst_hbm, g0_hbm, g1_hbm, dstloc, idxv, rows, sem):
    wid = lax.axis_index("s") * 2 + lax.axis_index("c")
    lane = lax.iota(jnp.int32, 16)
    pltpu.sync_copy(dst_hbm.at[pl.ds(wid * 2 * _CT, 2 * _CT)], dstloc)
    for ch in range(_CT // _CC):
        for slot in range(2):
            for j in range(_CC // 16):
                lt0 = ch * _CC + j * 16
                idxv[pl.ds(j * 16, 16)] = plsc.load_gather(
                    dstloc, [lane * 2 + (lt0 * 2 + slot)])
            pltpu.async_copy(o_hbm.at[idxv], rows, sem).wait()
            gh = g0_hbm if slot == 0 else g1_hbm
            pltpu.sync_copy(rows, gh.at[pl.ds(wid * _CT + ch * _CC, _CC)])


def _mlp_body(te_ref, ts_ref, x_ref, w1_ref, w3_ref, w2_ref, o_ref, xbf, xgs):
    t = pl.program_id(0)
    f = pl.program_id(1)

    @pl.when((t == 0) & (f == 0))
    def _():
        xbf[...] = x_ref[...].astype(jnp.bfloat16)

    @pl.when(f == 0)
    def _():
        p = (ts_ref[...] == lax.broadcasted_iota(jnp.int32, (BR, N), 1))
        xgs[...] = jnp.dot(p.astype(jnp.bfloat16), xbf[...],
                           preferred_element_type=jnp.float32
                           ).astype(jnp.bfloat16)
        o_ref[...] = jnp.zeros_like(o_ref)

    xb = xgs[...]
    w1b = w1_ref[0].astype(jnp.bfloat16)
    w3b = w3_ref[0].astype(jnp.bfloat16)
    a1 = jnp.dot(xb, w1b, preferred_element_type=jnp.float32)
    a3 = jnp.dot(xb, w3b, preferred_element_type=jnp.float32)
    h = (jax.nn.silu(a1) * a3).astype(jnp.bfloat16)
    w2b = w2_ref[0].astype(jnp.bfloat16)
    o_ref[...] += jnp.dot(h, w2b, preferred_element_type=jnp.float32)


def _grouped_mlp(te, tsrc2d, x, w1, w3, w2):
    grid_spec = pltpu.PrefetchScalarGridSpec(
        num_scalar_prefetch=1,
        grid=(T, NF),
        in_specs=[
            pl.BlockSpec((BR, 1), lambda t, f, te_r: (t, 0)),
            pl.BlockSpec((N, D), lambda t, f, te_r: (0, 0)),
            pl.BlockSpec((1, D, BF), lambda t, f, te_r: (te_r[t], 0, f)),
            pl.BlockSpec((1, D, BF), lambda t, f, te_r: (te_r[t], 0, f)),
            pl.BlockSpec((1, BF, D), lambda t, f, te_r: (te_r[t], f, 0)),
        ],
        out_specs=pl.BlockSpec((BR, D), lambda t, f, te_r: (t, 0)),
        scratch_shapes=[
            pltpu.VMEM((N, D), jnp.bfloat16),
            pltpu.VMEM((BR, D), jnp.bfloat16),
        ],
    )
    return pl.pallas_call(
        _mlp_body,
        grid_spec=grid_spec,
        out_shape=jax.ShapeDtypeStruct((CAP, D), jnp.float32),
        compiler_params=pltpu.CompilerParams(
            dimension_semantics=("arbitrary", "arbitrary")),
    )(te, tsrc2d, x, w1, w3, w2)


def _comb_body(g0_ref, g1_ref, e0_ref, e1_ref, y_ref):
    y_ref[...] = e0_ref[...] * g0_ref[...] + e1_ref[...] * g1_ref[...]


def _combine(g0, g1, ew):
    return pl.pallas_call(
        _comb_body,
        grid=(N // BR,),
        in_specs=[
            pl.BlockSpec((BR, D), lambda i: (i, 0)),
            pl.BlockSpec((BR, D), lambda i: (i, 0)),
            pl.BlockSpec((BR, 1), lambda i: (i, 0)),
            pl.BlockSpec((BR, 1), lambda i: (i, 0)),
        ],
        out_specs=pl.BlockSpec((BR, D), lambda i: (i, 0)),
        out_shape=jax.ShapeDtypeStruct((N, D), jnp.float32),
    )(g0, g1, ew[:, :1], ew[:, 1:])


def kernel(x, expert_weights, expert_indices, w1, w2, w3):
    eflat = expert_indices.reshape(-1).astype(jnp.int32)
    dst, tsrc, counts16, te = _sc_route(eflat)
    o = _grouped_mlp(te, tsrc.reshape(CAP, 1), x, w1, w3, w2)
    g0, g1 = _sc_pick(o, dst)
    y = _combine(g0, g1, expert_weights)
    return y, counts16[:8]
